# baseline (speedup 1.0000x reference)
import jax
import jax.numpy as jnp
from jax import lax
from jax.experimental import pallas as pl
from jax.experimental.pallas import tpu as pltpu
from jax.experimental.pallas import tpu_sc as plsc

_N = 10000
_E = 160000
_D = 256
_Q = 64

_NSUB = 16
_CHUNK = 125
_EPT = _E // _NSUB
_NCHUNK = _EPT // _CHUNK
_STRIPE = 1000

_mesh = plsc.VectorSubcoreMesh(
    core_axis_name="c", subcore_axis_name="s", num_cores=2, num_subcores=16
)


def _deg_body(src2, dst2, zrow, ones_h, dout, din, idx, ones_v, bounce,
              dsem, acc):
    cid = lax.axis_index("c")
    sid = lax.axis_index("s")
    @pl.when(sid < 10)
    def _():
        pltpu.sync_copy(zrow, bounce)
        pltpu.sync_copy(bounce, acc.at[pl.ds(sid * _STRIPE, _STRIPE)])

    pltpu.sync_copy(ones_h, ones_v)
    plsc.subcore_barrier()
    @pl.when(cid == 0)
    def _():
        pltpu.sync_copy(src2.at[pl.ds(sid * _NCHUNK, _NCHUNK)], idx)

    @pl.when(cid == 1)
    def _():
        pltpu.sync_copy(dst2.at[pl.ds(sid * _NCHUNK, _NCHUNK)], idx)

    def chunk(j, carry):
        pltpu.async_copy(ones_v, acc.at[idx.at[j]], dsem, add=True)

        @pl.when(j >= 4)
        def _():
            pltpu.make_async_copy(ones_v, acc.at[idx.at[j - 4]], dsem).wait()

        return carry

    lax.fori_loop(0, _NCHUNK, chunk, 0)
    for t in range(4):
        pltpu.make_async_copy(ones_v, acc.at[idx.at[_NCHUNK - 4 + t]],
                              dsem).wait()
    plsc.subcore_barrier()

    @pl.when(jnp.logical_and(cid == 0, sid < 10))
    def _():
        pltpu.sync_copy(acc.at[pl.ds(sid * _STRIPE, _STRIPE)], bounce)
        pltpu.sync_copy(bounce, dout.at[pl.ds(sid * _STRIPE, _STRIPE)])

    @pl.when(jnp.logical_and(cid == 1, sid < 10))
    def _():
        pltpu.sync_copy(acc.at[pl.ds(sid * _STRIPE, _STRIPE)], bounce)
        pltpu.sync_copy(bounce, din.at[pl.ds(sid * _STRIPE, _STRIPE)])


_deg_call = pl.kernel(
    _deg_body,
    out_type=(
        jax.ShapeDtypeStruct((_N,), jnp.float32),
        jax.ShapeDtypeStruct((_N,), jnp.float32),
    ),
    mesh=_mesh,
    scratch_types=[
        pltpu.VMEM((_NCHUNK, _CHUNK), jnp.int32),
        pltpu.VMEM((_CHUNK,), jnp.float32),
        pltpu.VMEM((_STRIPE,), jnp.float32),
        pltpu.SemaphoreType.DMA,
        pltpu.VMEM_SHARED((_N,), jnp.float32),
    ],
    compiler_params=pltpu.CompilerParams(use_tc_tiling_on_sc=False),
)


_NBUF = 4
_NH = _NBUF // 2


_RPT = _N // _NSUB
_ZR = 125


def _agg_body(hq0, hq1, hq2, hq3, src2, dst2, zq, aout,
              sidx, didx, rows, gsem, ssem, bounce, zbuf, acc):
    cid = lax.axis_index("c")
    sid = lax.axis_index("s")
    gs = [gsem.at[k] for k in range(_NBUF)]
    ss = [ssem.at[k] for k in range(_NBUF)]
    pltpu.sync_copy(src2.at[pl.ds(sid * _NCHUNK, _NCHUNK)], sidx)
    pltpu.sync_copy(dst2.at[pl.ds(sid * _NCHUNK, _NCHUNK)], didx)

    def scatter_all(table):
        bufs = [rows.at[k] for k in range(_NBUF)]
        for k in range(_NH):
            pltpu.async_copy(table.at[sidx.at[k]], bufs[k], gs[k])

        def group(g, carry):
            j0 = _NBUF * g
            for step in range(_NBUF):
                j = j0 + step
                pltpu.make_async_copy(table.at[sidx.at[j]], bufs[step],
                                      gs[step]).wait()
                pltpu.async_copy(bufs[step], acc.at[didx.at[j]], ss[step],
                                 add=True)
                pstep = (step - _NH) % _NBUF

                @pl.when(j >= _NH)
                def _():
                    pltpu.make_async_copy(bufs[pstep], acc.at[didx.at[j - _NH]],
                                          ss[pstep]).wait()

                @pl.when(j + _NH < _NCHUNK)
                def _():
                    pltpu.async_copy(table.at[sidx.at[j + _NH]], bufs[pstep],
                                     gs[pstep])

            return carry

        lax.fori_loop(0, _NCHUNK // _NBUF, group, 0)
        for t in range(_NH):
            j = _NCHUNK - _NH + t
            pltpu.make_async_copy(bufs[j % _NBUF], acc.at[didx.at[j]],
                                  ss[j % _NBUF]).wait()

    def copy_out(col, rezero):
        for k in range(_RPT // _ZR):
            off = sid * _RPT + k * _ZR
            pltpu.sync_copy(acc.at[pl.ds(off, _ZR)], bounce)
            pltpu.sync_copy(bounce,
                            aout.at[pl.ds(off, _ZR), pl.ds(col, _Q)])
            if rezero:
                pltpu.sync_copy(zbuf, acc.at[pl.ds(off, _ZR)])

    pltpu.sync_copy(zq, zbuf)
    for k in range(_RPT // _ZR):
        pltpu.sync_copy(zbuf, acc.at[pl.ds(sid * _RPT + k * _ZR, _ZR)])
    plsc.subcore_barrier()

    for phase, (t0, t1) in enumerate(((hq0, hq1), (hq2, hq3))):
        @pl.when(cid == 0)
        def _():
            scatter_all(t0)

        @pl.when(cid == 1)
        def _():
            scatter_all(t1)

        plsc.subcore_barrier()

        @pl.when(cid == 0)
        def _():
            copy_out(phase * 2 * _Q, phase == 0)

        @pl.when(cid == 1)
        def _():
            copy_out(phase * 2 * _Q + _Q, phase == 0)

        if phase == 0:
            plsc.subcore_barrier()


_agg_call = pl.kernel(
    _agg_body,
    out_type=jax.ShapeDtypeStruct((_N, _D), jnp.float32),
    mesh=_mesh,
    scratch_types=[
        pltpu.VMEM((_NCHUNK, _CHUNK), jnp.int32),
        pltpu.VMEM((_NCHUNK, _CHUNK), jnp.int32),
        pltpu.VMEM((_NBUF, _CHUNK, _Q), jnp.float32),
        pltpu.SemaphoreType.DMA((_NBUF,)),
        pltpu.SemaphoreType.DMA((_NBUF,)),
        pltpu.VMEM((_ZR, _Q), jnp.float32),
        pltpu.VMEM((_ZR, _Q), jnp.float32),
        pltpu.VMEM_SHARED((_N, _Q), jnp.float32),
    ],
    compiler_params=pltpu.CompilerParams(use_tc_tiling_on_sc=False),
)

_B = 1000


def _mm1_body(do_ref, di_ref, x_ref, w_ref, h_ref, ns_ref, nd_ref):
    do = do_ref[...]
    di = di_ref[...]
    ns = lax.rsqrt(jnp.where(do > 0, do, 1.0))
    nd = lax.rsqrt(jnp.where(di > 0, di, 1.0))
    x = x_ref[...] * ns
    h_ref[...] = jnp.dot(x, w_ref[...], preferred_element_type=jnp.float32)
    ns_ref[...] = ns
    nd_ref[...] = nd


_mm1_call = pl.pallas_call(
    _mm1_body,
    grid=(_N // _B,),
    in_specs=[
        pl.BlockSpec((_B, 1), lambda i: (i, 0)),
        pl.BlockSpec((_B, 1), lambda i: (i, 0)),
        pl.BlockSpec((_B, _D), lambda i: (i, 0)),
        pl.BlockSpec((_D, _D), lambda i: (0, 0)),
    ],
    out_specs=[
        pl.BlockSpec((_B, _D), lambda i: (i, 0)),
        pl.BlockSpec((_B, 1), lambda i: (i, 0)),
        pl.BlockSpec((_B, 1), lambda i: (i, 0)),
    ],
    out_shape=[
        jax.ShapeDtypeStruct((_N, _D), jnp.float32),
        jax.ShapeDtypeStruct((_N, 1), jnp.float32),
        jax.ShapeDtypeStruct((_N, 1), jnp.float32),
    ],
)


def _mm2_body(a_ref, ns_ref, nd_ref, b_ref, w_ref, h_ref):
    t = a_ref[...]
    t = jnp.maximum(t * nd_ref[...] + b_ref[...], 0.0) * ns_ref[...]
    h_ref[...] = jnp.dot(t, w_ref[...], preferred_element_type=jnp.float32)


_mm2_call = pl.pallas_call(
    _mm2_body,
    grid=(_N // _B,),
    in_specs=[pl.BlockSpec((_B, _D), lambda i: (i, 0))]
    + [
        pl.BlockSpec((_B, 1), lambda i: (i, 0)),
        pl.BlockSpec((_B, 1), lambda i: (i, 0)),
        pl.BlockSpec((1, _D), lambda i: (0, 0)),
        pl.BlockSpec((_D, _D), lambda i: (0, 0)),
    ],
    out_specs=pl.BlockSpec((_B, _D), lambda i: (i, 0)),
    out_shape=jax.ShapeDtypeStruct((_N, _D), jnp.float32),
)


def _mm3_body(a_ref, nd_ref, b_ref, w3_ref, b3_ref, out_ref):
    t = a_ref[...]
    t = jnp.maximum(t * nd_ref[...] + b_ref[...], 0.0)
    out_ref[...] = (
        jnp.dot(t, w3_ref[...], preferred_element_type=jnp.float32) + b3_ref[...]
    )


_mm3_call = pl.pallas_call(
    _mm3_body,
    grid=(_N // _B,),
    in_specs=[pl.BlockSpec((_B, _D), lambda i: (i, 0))]
    + [
        pl.BlockSpec((_B, 1), lambda i: (i, 0)),
        pl.BlockSpec((1, _D), lambda i: (0, 0)),
        pl.BlockSpec((_D, _Q), lambda i: (0, 0)),
        pl.BlockSpec((1, _Q), lambda i: (0, 0)),
    ],
    out_specs=pl.BlockSpec((_B, _Q), lambda i: (i, 0)),
    out_shape=jax.ShapeDtypeStruct((_N, _Q), jnp.float32),
)


@jax.jit
def kernel(features, edge_index, W1, b1, W2, b2, W3, b3):
    src2 = edge_index[0].reshape(_E // _CHUNK, _CHUNK)
    dst2 = edge_index[1].reshape(_E // _CHUNK, _CHUNK)
    zrow = jnp.zeros((_STRIPE,), jnp.float32)
    ones_h = jnp.ones((_CHUNK,), jnp.float32)
    zq = jnp.zeros((_ZR, _Q), jnp.float32)

    def quarters(h):
        return [h[:, k * _Q:(k + 1) * _Q] for k in range(4)]

    dout, din = _deg_call(src2, dst2, zrow, ones_h)
    h1, ns, nd = _mm1_call(dout.reshape(_N, 1), din.reshape(_N, 1),
                           features, W1)
    a1 = _agg_call(*quarters(h1), src2, dst2, zq)
    h2 = _mm2_call(a1, ns, nd, b1.reshape(1, _D), W2)
    a2 = _agg_call(*quarters(h2), src2, dst2, zq)
    logits = _mm3_call(a2, nd, b2.reshape(1, _D), W3, b3.reshape(1, _Q))
    return logits


# speedup vs baseline: 1.0414x; 1.0414x over previous
import jax
import jax.numpy as jnp
from jax import lax
from jax.experimental import pallas as pl
from jax.experimental.pallas import tpu as pltpu
from jax.experimental.pallas import tpu_sc as plsc

_N = 10000
_E = 160000
_D = 256
_Q = 64

_NSUB = 16
_CHUNK = 125
_EPT = _E // _NSUB
_NCHUNK = _EPT // _CHUNK
_STRIPE = 1000

_mesh = plsc.VectorSubcoreMesh(
    core_axis_name="c", subcore_axis_name="s", num_cores=2, num_subcores=16
)


def _deg_body(src2, dst2, zrow, ones_h, dout, din, idx, ones_v, bounce,
              dsem, acc):
    cid = lax.axis_index("c")
    sid = lax.axis_index("s")
    @pl.when(sid < 10)
    def _():
        pltpu.sync_copy(zrow, bounce)
        pltpu.sync_copy(bounce, acc.at[pl.ds(sid * _STRIPE, _STRIPE)])

    pltpu.sync_copy(ones_h, ones_v)
    plsc.subcore_barrier()
    @pl.when(cid == 0)
    def _():
        pltpu.sync_copy(src2.at[pl.ds(sid * _NCHUNK, _NCHUNK)], idx)

    @pl.when(cid == 1)
    def _():
        pltpu.sync_copy(dst2.at[pl.ds(sid * _NCHUNK, _NCHUNK)], idx)

    def chunk(j, carry):
        pltpu.async_copy(ones_v, acc.at[idx.at[j]], dsem, add=True)

        @pl.when(j >= 4)
        def _():
            pltpu.make_async_copy(ones_v, acc.at[idx.at[j - 4]], dsem).wait()

        return carry

    lax.fori_loop(0, _NCHUNK, chunk, 0)
    for t in range(4):
        pltpu.make_async_copy(ones_v, acc.at[idx.at[_NCHUNK - 4 + t]],
                              dsem).wait()
    plsc.subcore_barrier()

    @pl.when(jnp.logical_and(cid == 0, sid < 10))
    def _():
        pltpu.sync_copy(acc.at[pl.ds(sid * _STRIPE, _STRIPE)], bounce)
        pltpu.sync_copy(bounce, dout.at[pl.ds(sid * _STRIPE, _STRIPE)])

    @pl.when(jnp.logical_and(cid == 1, sid < 10))
    def _():
        pltpu.sync_copy(acc.at[pl.ds(sid * _STRIPE, _STRIPE)], bounce)
        pltpu.sync_copy(bounce, din.at[pl.ds(sid * _STRIPE, _STRIPE)])


_deg_call = pl.kernel(
    _deg_body,
    out_type=(
        jax.ShapeDtypeStruct((_N,), jnp.float32),
        jax.ShapeDtypeStruct((_N,), jnp.float32),
    ),
    mesh=_mesh,
    scratch_types=[
        pltpu.VMEM((_NCHUNK, _CHUNK), jnp.int32),
        pltpu.VMEM((_CHUNK,), jnp.float32),
        pltpu.VMEM((_STRIPE,), jnp.float32),
        pltpu.SemaphoreType.DMA,
        pltpu.VMEM_SHARED((_N,), jnp.float32),
    ],
    compiler_params=pltpu.CompilerParams(use_tc_tiling_on_sc=False),
)


_NBUF = 4
_NH = _NBUF // 2


_RPT = _N // _NSUB
_ZR = 125


def _agg_body(hq0, hq1, hq2, hq3, src2, dst2, zq, aout,
              sidx, didx, rows, gsem, ssem, bounce, zbuf, acc):
    cid = lax.axis_index("c")
    sid = lax.axis_index("s")
    gs = [gsem.at[k] for k in range(_NBUF)]
    ss = [ssem.at[k] for k in range(_NBUF)]
    pltpu.sync_copy(src2.at[pl.ds(sid * _NCHUNK, _NCHUNK)], sidx)
    pltpu.sync_copy(dst2.at[pl.ds(sid * _NCHUNK, _NCHUNK)], didx)

    def scatter_all(table):
        bufs = [rows.at[k] for k in range(_NBUF)]
        for k in range(_NH):
            pltpu.async_copy(table.at[sidx.at[k]], bufs[k], gs[k])

        def group(g, carry):
            j0 = _NBUF * g
            for step in range(_NBUF):
                j = j0 + step
                pltpu.make_async_copy(table.at[sidx.at[j]], bufs[step],
                                      gs[step]).wait()
                pltpu.async_copy(bufs[step], acc.at[didx.at[j]], ss[step],
                                 add=True)
                pstep = (step - _NH) % _NBUF

                @pl.when(j >= _NH)
                def _():
                    pltpu.make_async_copy(bufs[pstep], acc.at[didx.at[j - _NH]],
                                          ss[pstep]).wait()

                @pl.when(j + _NH < _NCHUNK)
                def _():
                    pltpu.async_copy(table.at[sidx.at[j + _NH]], bufs[pstep],
                                     gs[pstep])

            return carry

        lax.fori_loop(0, _NCHUNK // _NBUF, group, 0)
        for t in range(_NH):
            j = _NCHUNK - _NH + t
            pltpu.make_async_copy(bufs[j % _NBUF], acc.at[didx.at[j]],
                                  ss[j % _NBUF]).wait()

    def copy_out(col, rezero):
        for k in range(_RPT // _ZR):
            off = sid * _RPT + k * _ZR
            pltpu.sync_copy(acc.at[pl.ds(off, _ZR)], bounce)
            pltpu.sync_copy(bounce,
                            aout.at[pl.ds(off, _ZR), pl.ds(col, _Q)])
            if rezero:
                pltpu.sync_copy(zbuf, acc.at[pl.ds(off, _ZR)])

    pltpu.sync_copy(zq, zbuf)
    for k in range(_RPT // _ZR):
        pltpu.sync_copy(zbuf, acc.at[pl.ds(sid * _RPT + k * _ZR, _ZR)])
    plsc.subcore_barrier()

    for phase, (t0, t1) in enumerate(((hq0, hq1), (hq2, hq3))):
        @pl.when(cid == 0)
        def _():
            scatter_all(t0)

        @pl.when(cid == 1)
        def _():
            scatter_all(t1)

        plsc.subcore_barrier()

        @pl.when(cid == 0)
        def _():
            copy_out(phase * 2 * _Q, phase == 0)

        @pl.when(cid == 1)
        def _():
            copy_out(phase * 2 * _Q + _Q, phase == 0)

        if phase == 0:
            plsc.subcore_barrier()


_agg_call = pl.kernel(
    _agg_body,
    out_type=jax.ShapeDtypeStruct((_N, _D), jnp.float32),
    mesh=_mesh,
    scratch_types=[
        pltpu.VMEM((_NCHUNK, _CHUNK), jnp.int32),
        pltpu.VMEM((_NCHUNK, _CHUNK), jnp.int32),
        pltpu.VMEM((_NBUF, _CHUNK, _Q), jnp.float32),
        pltpu.SemaphoreType.DMA((_NBUF,)),
        pltpu.SemaphoreType.DMA((_NBUF,)),
        pltpu.VMEM((_ZR, _Q), jnp.float32),
        pltpu.VMEM((_ZR, _Q), jnp.float32),
        pltpu.VMEM_SHARED((_N, _Q), jnp.float32),
    ],
    compiler_params=pltpu.CompilerParams(use_tc_tiling_on_sc=False),
)

_B = 1000


def _mm1_body(do_ref, di_ref, x_ref, w_ref,
              h0_ref, h1_ref, h2_ref, h3_ref, ns_ref, nd_ref):
    do = do_ref[...]
    di = di_ref[...]
    ns = lax.rsqrt(jnp.where(do > 0, do, 1.0))
    nd = lax.rsqrt(jnp.where(di > 0, di, 1.0))
    x = x_ref[...] * ns
    h = jnp.dot(x, w_ref[...], preferred_element_type=jnp.float32)
    h0_ref[...] = h[:, 0 * _Q:1 * _Q]
    h1_ref[...] = h[:, 1 * _Q:2 * _Q]
    h2_ref[...] = h[:, 2 * _Q:3 * _Q]
    h3_ref[...] = h[:, 3 * _Q:4 * _Q]
    ns_ref[...] = ns
    nd_ref[...] = nd


_mm1_call = pl.pallas_call(
    _mm1_body,
    grid=(_N // _B,),
    in_specs=[
        pl.BlockSpec((_B, 1), lambda i: (i, 0)),
        pl.BlockSpec((_B, 1), lambda i: (i, 0)),
        pl.BlockSpec((_B, _D), lambda i: (i, 0)),
        pl.BlockSpec((_D, _D), lambda i: (0, 0)),
    ],
    out_specs=[pl.BlockSpec((_B, _Q), lambda i: (i, 0)) for _ in range(4)]
    + [
        pl.BlockSpec((_B, 1), lambda i: (i, 0)),
        pl.BlockSpec((_B, 1), lambda i: (i, 0)),
    ],
    out_shape=[jax.ShapeDtypeStruct((_N, _Q), jnp.float32) for _ in range(4)]
    + [
        jax.ShapeDtypeStruct((_N, 1), jnp.float32),
        jax.ShapeDtypeStruct((_N, 1), jnp.float32),
    ],
)


def _mm2_body(a_ref, ns_ref, nd_ref, b_ref, w_ref,
              h0_ref, h1_ref, h2_ref, h3_ref):
    t = a_ref[...]
    t = jnp.maximum(t * nd_ref[...] + b_ref[...], 0.0) * ns_ref[...]
    h = jnp.dot(t, w_ref[...], preferred_element_type=jnp.float32)
    h0_ref[...] = h[:, 0 * _Q:1 * _Q]
    h1_ref[...] = h[:, 1 * _Q:2 * _Q]
    h2_ref[...] = h[:, 2 * _Q:3 * _Q]
    h3_ref[...] = h[:, 3 * _Q:4 * _Q]


_mm2_call = pl.pallas_call(
    _mm2_body,
    grid=(_N // _B,),
    in_specs=[pl.BlockSpec((_B, _D), lambda i: (i, 0))]
    + [
        pl.BlockSpec((_B, 1), lambda i: (i, 0)),
        pl.BlockSpec((_B, 1), lambda i: (i, 0)),
        pl.BlockSpec((1, _D), lambda i: (0, 0)),
        pl.BlockSpec((_D, _D), lambda i: (0, 0)),
    ],
    out_specs=[pl.BlockSpec((_B, _Q), lambda i: (i, 0)) for _ in range(4)],
    out_shape=[jax.ShapeDtypeStruct((_N, _Q), jnp.float32) for _ in range(4)],
)


def _mm3_body(a_ref, nd_ref, b_ref, w3_ref, b3_ref, out_ref):
    t = a_ref[...]
    t = jnp.maximum(t * nd_ref[...] + b_ref[...], 0.0)
    out_ref[...] = (
        jnp.dot(t, w3_ref[...], preferred_element_type=jnp.float32) + b3_ref[...]
    )


_mm3_call = pl.pallas_call(
    _mm3_body,
    grid=(_N // _B,),
    in_specs=[pl.BlockSpec((_B, _D), lambda i: (i, 0))]
    + [
        pl.BlockSpec((_B, 1), lambda i: (i, 0)),
        pl.BlockSpec((1, _D), lambda i: (0, 0)),
        pl.BlockSpec((_D, _Q), lambda i: (0, 0)),
        pl.BlockSpec((1, _Q), lambda i: (0, 0)),
    ],
    out_specs=pl.BlockSpec((_B, _Q), lambda i: (i, 0)),
    out_shape=jax.ShapeDtypeStruct((_N, _Q), jnp.float32),
)


@jax.jit
def kernel(features, edge_index, W1, b1, W2, b2, W3, b3):
    src2 = edge_index[0].reshape(_E // _CHUNK, _CHUNK)
    dst2 = edge_index[1].reshape(_E // _CHUNK, _CHUNK)
    zrow = jnp.zeros((_STRIPE,), jnp.float32)
    ones_h = jnp.ones((_CHUNK,), jnp.float32)
    zq = jnp.zeros((_ZR, _Q), jnp.float32)

    dout, din = _deg_call(src2, dst2, zrow, ones_h)
    h1q = _mm1_call(dout.reshape(_N, 1), din.reshape(_N, 1), features, W1)
    h1, ns, nd = h1q[:4], h1q[4], h1q[5]
    a1 = _agg_call(*h1, src2, dst2, zq)
    h2 = _mm2_call(a1, ns, nd, b1.reshape(1, _D), W2)
    a2 = _agg_call(*h2, src2, dst2, zq)
    logits = _mm3_call(a2, nd, b2.reshape(1, _D), W3, b3.reshape(1, _Q))
    return logits
